# baseline (speedup 1.0000x reference)
import jax
import jax.numpy as jnp
from jax import lax
from jax.experimental import pallas as pl
from jax.experimental.pallas import tpu as pltpu
from jax.experimental.pallas import tpu_sc as plsc

N = 100000
R = 8
H = 32
C = 16
E = 1600000

NC, NS, L = 2, 16, 16
NW = NC * NS
K = 512
KR = K // 128
E1 = 1638400
EPW = E1 // NW
EPT = E1 // NS
NP = 100096
NPS = NP // NS

_mesh = plsc.VectorSubcoreMesh(core_axis_name="c", subcore_axis_name="s")
_f32 = jnp.float32
_i32 = jnp.int32
_params = pltpu.CompilerParams(use_tc_tiling_on_sc=False,
                               needs_layout_passes=False)

_GDN = lax.GatherDimensionNumbers(offset_dims=(), collapsed_slice_dims=(0,),
                                  start_index_map=(0,))


def _bcast_lane(v, k):
    idx = jnp.full((L, 1), k, _i32)
    return lax.gather(v, idx, _GDN, (1,),
                      mode=lax.GatherScatterMode.PROMISE_IN_BOUNDS)


def _gather_fire(table_h, idxref, rows, bi, sem):
    for j in range(KR):
        pltpu.async_copy(table_h.at[idxref.at[j]],
                         rows.at[bi, pl.ds(j * 128, 128)], sem)


def _gather_drain(table_h, idxref, rows, bi, sem):
    for j in range(KR):
        pltpu.make_async_copy(table_h.at[idxref.at[j]],
                              rows.at[bi, pl.ds(j * 128, 128)], sem).wait()


def _scatter_fire(rows, bi, acc, dsti, r, sem):
    for j in range(KR):
        pltpu.async_copy(rows.at[bi, pl.ds(j * 128, 128)],
                         acc.at[dsti.at[r, j]], sem, add=True)


def _scatter_drain(rows, bi, acc, dsti, r, sem):
    for j in range(KR):
        pltpu.make_async_copy(rows.at[bi, pl.ds(j * 128, 128)],
                              acc.at[dsti.at[r, j]], sem).wait()


def _zero_and_barrier(zeros_h, acc, s):
    pltpu.sync_copy(zeros_h, acc.at[pl.ds(s * NPS, NPS)])
    plsc.subcore_barrier()


def _writeout(acc, c, s, out0_h, out1_h):
    plsc.subcore_barrier()

    @pl.when(c == 0)
    def _():
        pltpu.sync_copy(acc.at[pl.ds(s * NPS, NPS)],
                        out0_h.at[pl.ds(s * NPS, NPS)])

    @pl.when(c == 1)
    def _():
        pltpu.sync_copy(acc.at[pl.ds(s * NPS, NPS)],
                        out1_h.at[pl.ds(s * NPS, NPS)])


def _count_body(dst_h, rel_h, zeros_h, out0_h, out1_h, acc, dsti, reli, rows):
    c = lax.axis_index("c")
    s = lax.axis_index("s")
    _zero_and_barrier(zeros_h, acc, s)
    wid = s * NC + c
    iot = lax.broadcasted_iota(_i32, (L,), 0)

    def chunk(i, _):
        rb = wid * (EPW // 128) + i * KR
        pltpu.sync_copy(dst_h.at[pl.ds(rb, KR)], dsti)
        pltpu.sync_copy(rel_h.at[pl.ds(rb, KR)], reli)

        def og(g, _):
            def ol(l, _):
                eb = g * 128 + l * 16
                relv = reli[g, pl.ds(l * 16, 16)]
                for k in range(16):
                    rows[eb + k] = jnp.where(iot == relv[k], 1.0, 0.0)
                return 0
            return lax.fori_loop(0, 8, ol, 0)
        lax.fori_loop(0, KR, og, 0)
        for j in range(KR):
            pltpu.sync_copy(rows.at[pl.ds(j * 128, 128)],
                            acc.at[dsti.at[j]], add=True)
        return 0

    lax.fori_loop(0, EPW // K, chunk, 0)
    _writeout(acc, c, s, out0_h, out1_h)


def _norm_body(dst_h, rel_h, cnt0_h, cnt1_h, out_h, dsti, reli, rows0, rows1,
               nout, gsem, isem, osem):
    c = lax.axis_index("c")
    s = lax.axis_index("s")
    wid = s * NC + c
    nch = EPW // K
    base_rb = wid * (EPW // 128)
    iot = lax.broadcasted_iota(_i32, (L,), 0)

    def idx_fire(i, r, sync=False):
        rb = base_rb + i * KR
        cp = pltpu.sync_copy if sync else (
            lambda a, b: pltpu.async_copy(a, b, isem))
        cp(dst_h.at[pl.ds(rb, KR)], dsti.at[r])
        cp(rel_h.at[pl.ds(rb, KR)], reli.at[r])

    def idx_drain(i, r):
        rb = base_rb + i * KR
        pltpu.make_async_copy(dst_h.at[pl.ds(rb, KR)], dsti.at[r], isem).wait()
        pltpu.make_async_copy(rel_h.at[pl.ds(rb, KR)], reli.at[r], isem).wait()

    def fire2(r, bi):
        _gather_fire(cnt0_h, dsti.at[r], rows0, bi, gsem)
        _gather_fire(cnt1_h, dsti.at[r], rows1, bi, gsem)

    def drain2(r, bi):
        _gather_drain(cnt0_h, dsti.at[r], rows0, bi, gsem)
        _gather_drain(cnt1_h, dsti.at[r], rows1, bi, gsem)

    def permute(b, bi):
        bvec = jnp.full((L,), bi, _i32)

        def pg(g, _):
            def plp(l, _):
                eb = g * 128 + l * 16
                relv = reli[b, g, pl.ds(l * 16, 16)]
                nv0 = plsc.load_gather(rows0, [bvec, eb + iot, relv])
                nv1 = plsc.load_gather(rows1, [bvec, eb + iot, relv])
                nout[bi, g, pl.ds(l * 16, 16)] = (
                    1.0 / jnp.maximum(nv0 + nv1, 1.0))
                return 0
            return lax.fori_loop(0, 8, plp, 0)
        lax.fori_loop(0, KR, pg, 0)

    idx_fire(0, 0, sync=True)
    idx_fire(1, 1)
    fire2(0, 0)

    def outer(g, _):
        i0 = g * 4
        for b in range(4):
            i = i0 + b
            bi = b % 2

            @pl.when(i + 2 < nch)
            def _():
                idx_fire(i + 2, (b + 2) % 4)
            drain2(b, bi)

            @pl.when(i >= 1)
            def _():
                pltpu.make_async_copy(
                    nout.at[1 - bi],
                    out_h.at[pl.ds(base_rb + (i - 1) * KR, KR)], osem).wait()

            @pl.when(i + 1 < nch)
            def _():
                idx_drain(i + 1, (b + 1) % 4)
                fire2((b + 1) % 4, 1 - bi)
            permute(b, bi)
            pltpu.async_copy(nout.at[bi],
                             out_h.at[pl.ds(base_rb + i * KR, KR)], osem)
        return 0

    lax.fori_loop(0, nch // 4, outer, 0)
    pltpu.make_async_copy(nout.at[1],
                          out_h.at[pl.ds(base_rb + (nch - 1) * KR, KR)],
                          osem).wait()


def _edge_pass(src_h, rel_h, dst_h, nrm_h, tab_h, zeros_h, out0_h, out1_h,
               acc, srci, reli, dsti, nrmi, tidx, rows,
               gsem, isem, ssem, nch, base_rb, make_idx):
    c = lax.axis_index("c")
    s = lax.axis_index("s")
    _zero_and_barrier(zeros_h, acc, s)

    def idx_fire(i, r, sync=False):
        rb = base_rb + i * KR
        cp = pltpu.sync_copy if sync else (
            lambda a, b: pltpu.async_copy(a, b, isem))
        cp(src_h.at[pl.ds(rb, KR)], srci.at[r])
        cp(rel_h.at[pl.ds(rb, KR)], reli.at[r])
        cp(dst_h.at[pl.ds(rb, KR)], dsti.at[r])
        cp(nrm_h.at[pl.ds(rb, KR)], nrmi.at[r])

    def idx_drain(i, r):
        rb = base_rb + i * KR
        pltpu.make_async_copy(src_h.at[pl.ds(rb, KR)], srci.at[r], isem).wait()
        pltpu.make_async_copy(rel_h.at[pl.ds(rb, KR)], reli.at[r], isem).wait()
        pltpu.make_async_copy(dst_h.at[pl.ds(rb, KR)], dsti.at[r], isem).wait()
        pltpu.make_async_copy(nrm_h.at[pl.ds(rb, KR)], nrmi.at[r], isem).wait()

    def compute_tidx(r):
        def ig(g, _):
            def il(l, _):
                sv = srci[r, g, pl.ds(l * 16, 16)]
                rv = reli[r, g, pl.ds(l * 16, 16)]
                tidx[g, pl.ds(l * 16, 16)] = make_idx(sv, rv, c)
                return 0
            return lax.fori_loop(0, 8, il, 0)
        lax.fori_loop(0, KR, ig, 0)

    def scale(b, bi):
        def sg(g, _):
            def sl(l, _):
                eb = g * 128 + l * 16
                nv = nrmi[b, g, pl.ds(l * 16, 16)]
                for k in range(16):
                    rows[bi, eb + k] = rows[bi, eb + k] * _bcast_lane(nv, k)
                return 0
            return lax.fori_loop(0, 8, sl, 0)
        lax.fori_loop(0, KR, sg, 0)

    idx_fire(0, 0, sync=True)
    idx_fire(1, 1)
    compute_tidx(0)
    _gather_fire(tab_h, tidx, rows, 0, gsem)

    def outer(g, _):
        i0 = g * 4
        for b in range(4):
            i = i0 + b
            bi = b % 2

            @pl.when(i + 2 < nch)
            def _():
                idx_fire(i + 2, (b + 2) % 4)
            _gather_drain(tab_h, tidx, rows, bi, gsem)

            @pl.when(i >= 1)
            def _():
                _scatter_drain(rows, 1 - bi, acc, dsti, (b + 3) % 4, ssem)

            @pl.when(i + 1 < nch)
            def _():
                idx_drain(i + 1, (b + 1) % 4)
                compute_tidx((b + 1) % 4)
                _gather_fire(tab_h, tidx, rows, 1 - bi, gsem)
            scale(b, bi)
            _scatter_fire(rows, bi, acc, dsti, b, ssem)
        return 0

    lax.fori_loop(0, nch // 4, outer, 0)
    _scatter_drain(rows, 1, acc, dsti, 3, ssem)
    _writeout(acc, c, s, out0_h, out1_h)


def _layer1_body(src_h, rel_h, dst_h, nrm_h, w1_h, zeros_h, out0_h, out1_h,
                 acc, srci, reli, dsti, nrmi, tidx, rows, gsem, isem, ssem):
    s = lax.axis_index("s")
    _edge_pass(src_h, rel_h, dst_h, nrm_h, w1_h, zeros_h, out0_h, out1_h,
               acc, srci, reli, dsti, nrmi, tidx, rows, gsem, isem, ssem,
               nch=EPT // K, base_rb=s * (EPT // 128),
               make_idx=lambda sv, rv, c: (rv * N + sv) * 2 + c)


def _layer2_body(src_h, rel_h, dst_h, nrm_h, z_h, zeros_h, out0_h, out1_h,
                 acc, srci, reli, dsti, nrmi, tidx, rows, gsem, isem, ssem):
    c = lax.axis_index("c")
    s = lax.axis_index("s")
    wid = s * NC + c
    _edge_pass(src_h, rel_h, dst_h, nrm_h, z_h, zeros_h, out0_h, out1_h,
               acc, srci, reli, dsti, nrmi, tidx, rows, gsem, isem, ssem,
               nch=EPW // K, base_rb=wid * (EPW // 128),
               make_idx=lambda sv, rv, c: sv * R + rv)


_EDGE_SCRATCH = [
    pltpu.VMEM_SHARED((NP, L), _f32),
    pltpu.VMEM((4, KR, 128), _i32),
    pltpu.VMEM((4, KR, 128), _i32),
    pltpu.VMEM((4, KR, 128), _i32),
    pltpu.VMEM((4, KR, 128), _f32),
    pltpu.VMEM((KR, 128), _i32),
    pltpu.VMEM((2, K, L), _f32),
    pltpu.SemaphoreType.DMA,
    pltpu.SemaphoreType.DMA,
    pltpu.SemaphoreType.DMA,
]

_PAIR_OUT = [jax.ShapeDtypeStruct((NP, L), _f32),
             jax.ShapeDtypeStruct((NP, L), _f32)]


def _sc_count(dst2, rel2, zeros2):
    return pl.kernel(
        _count_body, out_type=_PAIR_OUT, mesh=_mesh, compiler_params=_params,
        scratch_types=[
            pltpu.VMEM_SHARED((NP, L), _f32),
            pltpu.VMEM((KR, 128), _i32),
            pltpu.VMEM((KR, 128), _i32),
            pltpu.VMEM((K, L), _f32),
        ],
    )(dst2, rel2, zeros2)


def _sc_norm(dst2, rel2, cnt0, cnt1):
    return pl.kernel(
        _norm_body, out_type=jax.ShapeDtypeStruct((E1 // 128, 128), _f32),
        mesh=_mesh, compiler_params=_params,
        scratch_types=[
            pltpu.VMEM((4, KR, 128), _i32),
            pltpu.VMEM((4, KR, 128), _i32),
            pltpu.VMEM((2, K, L), _f32),
            pltpu.VMEM((2, K, L), _f32),
            pltpu.VMEM((2, KR, 128), _f32),
            pltpu.SemaphoreType.DMA,
            pltpu.SemaphoreType.DMA,
            pltpu.SemaphoreType.DMA,
        ],
    )(dst2, rel2, cnt0, cnt1)


def _sc_layer1(src2, rel2, dst2, nrm2, w1r, zeros2):
    return pl.kernel(
        _layer1_body, out_type=_PAIR_OUT, mesh=_mesh, compiler_params=_params,
        scratch_types=_EDGE_SCRATCH,
    )(src2, rel2, dst2, nrm2, w1r, zeros2)


def _sc_layer2(src2, rel2, dst2, nrm2, zrows, zeros2):
    return pl.kernel(
        _layer2_body, out_type=_PAIR_OUT, mesh=_mesh, compiler_params=_params,
        scratch_types=_EDGE_SCRATCH,
    )(src2, rel2, dst2, nrm2, zrows, zeros2)


def _tc_dense_body(a0, a1, r1, b1, w2, r2, b2, z_o, xr_o):
    x = jnp.concatenate([a0[...], a1[...]], axis=1)
    x = jax.nn.relu(x + r1[...] + b1[...])
    z_o[...] = jnp.dot(x, w2[...], preferred_element_type=_f32)
    xr_o[...] = jnp.dot(x, r2[...], preferred_element_type=_f32) + b2[...]


def _tc_out_body(p0, p1, xr, o):
    o[...] = jax.nn.sigmoid(p0[...] + p1[...] + xr[...])


def kernel(edge_index, edge_type, weight1, root1, bias1, weight2, root2, bias2):
    src = edge_index[0].astype(_i32)
    dst = edge_index[1].astype(_i32)
    rel = edge_type.astype(_i32)
    pad = E1 - E
    src2 = jnp.concatenate([src, jnp.zeros((pad,), _i32)]).reshape(E1 // 128, 128)
    dst2 = jnp.concatenate([dst, jnp.full((pad,), N, _i32)]).reshape(E1 // 128, 128)
    rel2 = jnp.concatenate([rel, jnp.zeros((pad,), _i32)]).reshape(E1 // 128, 128)
    zeros2 = jnp.zeros((NPS, L), _f32)

    cnt0, cnt1 = _sc_count(dst2, rel2, zeros2)

    nrm2 = _sc_norm(dst2, rel2, cnt0, cnt1)

    w1r = weight1.reshape(R * N * 2, 16)
    a10, a11 = _sc_layer1(src2, rel2, dst2, nrm2, w1r, zeros2)

    w2cat = jnp.transpose(weight2, (1, 0, 2)).reshape(H, R * C)
    blk = 1000
    grid_e = N // blk
    z, xr = pl.pallas_call(
        _tc_dense_body,
        out_shape=[jax.ShapeDtypeStruct((N, R * C), _f32),
                   jax.ShapeDtypeStruct((N, C), _f32)],
        grid=(grid_e,),
        in_specs=[pl.BlockSpec((blk, L), lambda i: (i, 0)),
                  pl.BlockSpec((blk, L), lambda i: (i, 0)),
                  pl.BlockSpec((blk, H), lambda i: (i, 0)),
                  pl.BlockSpec((1, H), lambda i: (0, 0)),
                  pl.BlockSpec((H, R * C), lambda i: (0, 0)),
                  pl.BlockSpec((H, C), lambda i: (0, 0)),
                  pl.BlockSpec((1, C), lambda i: (0, 0))],
        out_specs=[pl.BlockSpec((blk, R * C), lambda i: (i, 0)),
                   pl.BlockSpec((blk, C), lambda i: (i, 0))],
    )(a10, a11, root1, bias1.reshape(1, H), w2cat, root2, bias2.reshape(1, C))
    zrows = z.reshape(N * R, C)

    p0, p1 = _sc_layer2(src2, rel2, dst2, nrm2, zrows, zeros2)

    blk_g = 1000
    grid_g = N // blk_g
    return pl.pallas_call(
        _tc_out_body,
        out_shape=jax.ShapeDtypeStruct((N, C), _f32),
        grid=(grid_g,),
        in_specs=[pl.BlockSpec((blk_g, C), lambda i: (i, 0)),
                  pl.BlockSpec((blk_g, C), lambda i: (i, 0)),
                  pl.BlockSpec((blk_g, C), lambda i: (i, 0))],
        out_specs=pl.BlockSpec((blk_g, C), lambda i: (i, 0)),
    )(p0, p1, xr)


# speedup vs baseline: 1.0125x; 1.0125x over previous
import jax
import jax.numpy as jnp
from jax import lax
from jax.experimental import pallas as pl
from jax.experimental.pallas import tpu as pltpu
from jax.experimental.pallas import tpu_sc as plsc

N = 100000
R = 8
H = 32
C = 16
E = 1600000

NC, NS, L = 2, 16, 16
NW = NC * NS
K = 512
KR = K // 128
E1 = 1638400
EPW = E1 // NW
EPT = E1 // NS
NP = 100096
NPS = NP // NS

_mesh = plsc.VectorSubcoreMesh(core_axis_name="c", subcore_axis_name="s")
_f32 = jnp.float32
_i32 = jnp.int32
_params = pltpu.CompilerParams(use_tc_tiling_on_sc=False,
                               needs_layout_passes=False)

_GDN = lax.GatherDimensionNumbers(offset_dims=(), collapsed_slice_dims=(0,),
                                  start_index_map=(0,))


def _bcast_lane(v, k):
    idx = jnp.full((L, 1), k, _i32)
    return lax.gather(v, idx, _GDN, (1,),
                      mode=lax.GatherScatterMode.PROMISE_IN_BOUNDS)


def _gather_fire(table_h, idxref, rows, bi, sem):
    for j in range(KR):
        pltpu.async_copy(table_h.at[idxref.at[j]],
                         rows.at[bi, pl.ds(j * 128, 128)], sem)


def _gather_drain(table_h, idxref, rows, bi, sem):
    for j in range(KR):
        pltpu.make_async_copy(table_h.at[idxref.at[j]],
                              rows.at[bi, pl.ds(j * 128, 128)], sem).wait()


def _scatter_fire(rows, bi, acc, dsti, r, sem):
    for j in range(KR):
        pltpu.async_copy(rows.at[bi, pl.ds(j * 128, 128)],
                         acc.at[dsti.at[r, j]], sem, add=True)


def _scatter_drain(rows, bi, acc, dsti, r, sem):
    for j in range(KR):
        pltpu.make_async_copy(rows.at[bi, pl.ds(j * 128, 128)],
                              acc.at[dsti.at[r, j]], sem).wait()


def _zero_and_barrier(zeros_h, acc, s):
    pltpu.sync_copy(zeros_h, acc.at[pl.ds(s * NPS, NPS)])
    plsc.subcore_barrier()


def _writeout(acc, c, s, out0_h, out1_h):
    plsc.subcore_barrier()

    @pl.when(c == 0)
    def _():
        pltpu.sync_copy(acc.at[pl.ds(s * NPS, NPS)],
                        out0_h.at[pl.ds(s * NPS, NPS)])

    @pl.when(c == 1)
    def _():
        pltpu.sync_copy(acc.at[pl.ds(s * NPS, NPS)],
                        out1_h.at[pl.ds(s * NPS, NPS)])


def _count_body(dst_h, rel_h, zeros_h, out0_h, out1_h, acc, dsti, reli, rows):
    c = lax.axis_index("c")
    s = lax.axis_index("s")
    _zero_and_barrier(zeros_h, acc, s)
    wid = s * NC + c
    iot = lax.broadcasted_iota(_i32, (L,), 0)

    def chunk(i, _):
        rb = wid * (EPW // 128) + i * KR
        pltpu.sync_copy(dst_h.at[pl.ds(rb, KR)], dsti)
        pltpu.sync_copy(rel_h.at[pl.ds(rb, KR)], reli)

        def og(g, _):
            def ol(l, _):
                eb = g * 128 + l * 16
                relv = reli[g, pl.ds(l * 16, 16)]
                for k in range(16):
                    rows[eb + k] = jnp.where(iot == relv[k], 1.0, 0.0)
                return 0
            return lax.fori_loop(0, 8, ol, 0)
        lax.fori_loop(0, KR, og, 0)
        for j in range(KR):
            pltpu.sync_copy(rows.at[pl.ds(j * 128, 128)],
                            acc.at[dsti.at[j]], add=True)
        return 0

    lax.fori_loop(0, EPW // K, chunk, 0)
    _writeout(acc, c, s, out0_h, out1_h)


def _norm_body(dst_h, rel_h, nrow_h, out_h, dsti, reli, rows0,
               nout, gsem, isem, osem):
    c = lax.axis_index("c")
    s = lax.axis_index("s")
    wid = s * NC + c
    nch = EPW // K
    base_rb = wid * (EPW // 128)
    iot = lax.broadcasted_iota(_i32, (L,), 0)

    def idx_fire(i, r, sync=False):
        rb = base_rb + i * KR
        cp = pltpu.sync_copy if sync else (
            lambda a, b: pltpu.async_copy(a, b, isem))
        cp(dst_h.at[pl.ds(rb, KR)], dsti.at[r])
        cp(rel_h.at[pl.ds(rb, KR)], reli.at[r])

    def idx_drain(i, r):
        rb = base_rb + i * KR
        pltpu.make_async_copy(dst_h.at[pl.ds(rb, KR)], dsti.at[r], isem).wait()
        pltpu.make_async_copy(rel_h.at[pl.ds(rb, KR)], reli.at[r], isem).wait()

    def fire2(r, bi):
        _gather_fire(nrow_h, dsti.at[r], rows0, bi, gsem)

    def drain2(r, bi):
        _gather_drain(nrow_h, dsti.at[r], rows0, bi, gsem)

    def permute(b, bi):
        bvec = jnp.full((L,), bi, _i32)

        def pg(g, _):
            def plp(l, _):
                eb = g * 128 + l * 16
                relv = reli[b, g, pl.ds(l * 16, 16)]
                nv = plsc.load_gather(rows0, [bvec, eb + iot, relv])
                nout[bi, g, pl.ds(l * 16, 16)] = nv
                return 0
            return lax.fori_loop(0, 8, plp, 0)
        lax.fori_loop(0, KR, pg, 0)

    idx_fire(0, 0, sync=True)
    idx_fire(1, 1)
    fire2(0, 0)

    def outer(g, _):
        i0 = g * 4
        for b in range(4):
            i = i0 + b
            bi = b % 2

            @pl.when(i + 2 < nch)
            def _():
                idx_fire(i + 2, (b + 2) % 4)
            drain2(b, bi)

            @pl.when(i >= 1)
            def _():
                pltpu.make_async_copy(
                    nout.at[1 - bi],
                    out_h.at[pl.ds(base_rb + (i - 1) * KR, KR)], osem).wait()

            @pl.when(i + 1 < nch)
            def _():
                idx_drain(i + 1, (b + 1) % 4)
                fire2((b + 1) % 4, 1 - bi)
            permute(b, bi)
            pltpu.async_copy(nout.at[bi],
                             out_h.at[pl.ds(base_rb + i * KR, KR)], osem)
        return 0

    lax.fori_loop(0, nch // 4, outer, 0)
    pltpu.make_async_copy(nout.at[1],
                          out_h.at[pl.ds(base_rb + (nch - 1) * KR, KR)],
                          osem).wait()


def _edge_pass(src_h, rel_h, dst_h, nrm_h, tab_h, zeros_h, out0_h, out1_h,
               acc, srci, reli, dsti, nrmi, tidx, rows,
               gsem, isem, ssem, nch, base_rb, make_idx):
    c = lax.axis_index("c")
    s = lax.axis_index("s")
    _zero_and_barrier(zeros_h, acc, s)

    def idx_fire(i, r, sync=False):
        rb = base_rb + i * KR
        cp = pltpu.sync_copy if sync else (
            lambda a, b: pltpu.async_copy(a, b, isem))
        cp(src_h.at[pl.ds(rb, KR)], srci.at[r])
        cp(rel_h.at[pl.ds(rb, KR)], reli.at[r])
        cp(dst_h.at[pl.ds(rb, KR)], dsti.at[r])
        cp(nrm_h.at[pl.ds(rb, KR)], nrmi.at[r])

    def idx_drain(i, r):
        rb = base_rb + i * KR
        pltpu.make_async_copy(src_h.at[pl.ds(rb, KR)], srci.at[r], isem).wait()
        pltpu.make_async_copy(rel_h.at[pl.ds(rb, KR)], reli.at[r], isem).wait()
        pltpu.make_async_copy(dst_h.at[pl.ds(rb, KR)], dsti.at[r], isem).wait()
        pltpu.make_async_copy(nrm_h.at[pl.ds(rb, KR)], nrmi.at[r], isem).wait()

    def compute_tidx(r):
        def ig(g, _):
            def il(l, _):
                sv = srci[r, g, pl.ds(l * 16, 16)]
                rv = reli[r, g, pl.ds(l * 16, 16)]
                tidx[g, pl.ds(l * 16, 16)] = make_idx(sv, rv, c)
                return 0
            return lax.fori_loop(0, 8, il, 0)
        lax.fori_loop(0, KR, ig, 0)

    def scale(b, bi):
        def sg(g, _):
            def sl(l, _):
                eb = g * 128 + l * 16
                nv = nrmi[b, g, pl.ds(l * 16, 16)]
                for k in range(16):
                    rows[bi, eb + k] = rows[bi, eb + k] * _bcast_lane(nv, k)
                return 0
            return lax.fori_loop(0, 8, sl, 0)
        lax.fori_loop(0, KR, sg, 0)

    idx_fire(0, 0, sync=True)
    idx_fire(1, 1)
    compute_tidx(0)
    _gather_fire(tab_h, tidx, rows, 0, gsem)

    def outer(g, _):
        i0 = g * 4
        for b in range(4):
            i = i0 + b
            bi = b % 2

            @pl.when(i + 2 < nch)
            def _():
                idx_fire(i + 2, (b + 2) % 4)
            _gather_drain(tab_h, tidx, rows, bi, gsem)

            @pl.when(i >= 1)
            def _():
                _scatter_drain(rows, 1 - bi, acc, dsti, (b + 3) % 4, ssem)

            @pl.when(i + 1 < nch)
            def _():
                idx_drain(i + 1, (b + 1) % 4)
                compute_tidx((b + 1) % 4)
                _gather_fire(tab_h, tidx, rows, 1 - bi, gsem)
            scale(b, bi)
            _scatter_fire(rows, bi, acc, dsti, b, ssem)
        return 0

    lax.fori_loop(0, nch // 4, outer, 0)
    _scatter_drain(rows, 1, acc, dsti, 3, ssem)
    _writeout(acc, c, s, out0_h, out1_h)


def _layer1_body(src_h, rel_h, dst_h, nrm_h, w1_h, zeros_h, out0_h, out1_h,
                 acc, srci, reli, dsti, nrmi, tidx, rows, gsem, isem, ssem):
    s = lax.axis_index("s")
    _edge_pass(src_h, rel_h, dst_h, nrm_h, w1_h, zeros_h, out0_h, out1_h,
               acc, srci, reli, dsti, nrmi, tidx, rows, gsem, isem, ssem,
               nch=EPT // K, base_rb=s * (EPT // 128),
               make_idx=lambda sv, rv, c: (rv * N + sv) * 2 + c)


def _layer2_body(src_h, rel_h, dst_h, nrm_h, z_h, zeros_h, out0_h, out1_h,
                 acc, srci, reli, dsti, nrmi, tidx, rows, gsem, isem, ssem):
    c = lax.axis_index("c")
    s = lax.axis_index("s")
    wid = s * NC + c
    _edge_pass(src_h, rel_h, dst_h, nrm_h, z_h, zeros_h, out0_h, out1_h,
               acc, srci, reli, dsti, nrmi, tidx, rows, gsem, isem, ssem,
               nch=EPW // K, base_rb=wid * (EPW // 128),
               make_idx=lambda sv, rv, c: sv * R + rv)


_EDGE_SCRATCH = [
    pltpu.VMEM_SHARED((NP, L), _f32),
    pltpu.VMEM((4, KR, 128), _i32),
    pltpu.VMEM((4, KR, 128), _i32),
    pltpu.VMEM((4, KR, 128), _i32),
    pltpu.VMEM((4, KR, 128), _f32),
    pltpu.VMEM((KR, 128), _i32),
    pltpu.VMEM((2, K, L), _f32),
    pltpu.SemaphoreType.DMA,
    pltpu.SemaphoreType.DMA,
    pltpu.SemaphoreType.DMA,
]

_PAIR_OUT = [jax.ShapeDtypeStruct((NP, L), _f32),
             jax.ShapeDtypeStruct((NP, L), _f32)]


def _sc_count(dst2, rel2, zeros2):
    return pl.kernel(
        _count_body, out_type=_PAIR_OUT, mesh=_mesh, compiler_params=_params,
        scratch_types=[
            pltpu.VMEM_SHARED((NP, L), _f32),
            pltpu.VMEM((KR, 128), _i32),
            pltpu.VMEM((KR, 128), _i32),
            pltpu.VMEM((K, L), _f32),
        ],
    )(dst2, rel2, zeros2)


def _sc_norm(dst2, rel2, nrow):
    return pl.kernel(
        _norm_body, out_type=jax.ShapeDtypeStruct((E1 // 128, 128), _f32),
        mesh=_mesh, compiler_params=_params,
        scratch_types=[
            pltpu.VMEM((4, KR, 128), _i32),
            pltpu.VMEM((4, KR, 128), _i32),
            pltpu.VMEM((2, K, L), _f32),
            pltpu.VMEM((2, KR, 128), _f32),
            pltpu.SemaphoreType.DMA,
            pltpu.SemaphoreType.DMA,
            pltpu.SemaphoreType.DMA,
        ],
    )(dst2, rel2, nrow)


def _sc_layer1(src2, rel2, dst2, nrm2, w1r, zeros2):
    return pl.kernel(
        _layer1_body, out_type=_PAIR_OUT, mesh=_mesh, compiler_params=_params,
        scratch_types=_EDGE_SCRATCH,
    )(src2, rel2, dst2, nrm2, w1r, zeros2)


def _sc_layer2(src2, rel2, dst2, nrm2, zrows, zeros2):
    return pl.kernel(
        _layer2_body, out_type=_PAIR_OUT, mesh=_mesh, compiler_params=_params,
        scratch_types=_EDGE_SCRATCH,
    )(src2, rel2, dst2, nrm2, zrows, zeros2)


def _tc_norm_body(p0, p1, o):
    o[...] = 1.0 / jnp.maximum(p0[...] + p1[...], 1.0)


def _tc_dense_body(a0, a1, r1, b1, w2, r2, b2, z_o, xr_o):
    x = jnp.concatenate([a0[...], a1[...]], axis=1)
    x = jax.nn.relu(x + r1[...] + b1[...])
    z_o[...] = jnp.dot(x, w2[...], preferred_element_type=_f32)
    xr_o[...] = jnp.dot(x, r2[...], preferred_element_type=_f32) + b2[...]


def _tc_out_body(p0, p1, xr, o):
    o[...] = jax.nn.sigmoid(p0[...] + p1[...] + xr[...])


def kernel(edge_index, edge_type, weight1, root1, bias1, weight2, root2, bias2):
    src = edge_index[0].astype(_i32)
    dst = edge_index[1].astype(_i32)
    rel = edge_type.astype(_i32)
    pad = E1 - E
    src2 = jnp.concatenate([src, jnp.zeros((pad,), _i32)]).reshape(E1 // 128, 128)
    dst2 = jnp.concatenate([dst, jnp.full((pad,), N, _i32)]).reshape(E1 // 128, 128)
    rel2 = jnp.concatenate([rel, jnp.zeros((pad,), _i32)]).reshape(E1 // 128, 128)
    zeros2 = jnp.zeros((NPS, L), _f32)

    cnt0, cnt1 = _sc_count(dst2, rel2, zeros2)

    blk_b = 4352
    grid_b = NP // blk_b
    nrow = pl.pallas_call(
        _tc_norm_body,
        out_shape=jax.ShapeDtypeStruct((NP, L), _f32),
        grid=(grid_b,),
        in_specs=[pl.BlockSpec((blk_b, L), lambda i: (i, 0)),
                  pl.BlockSpec((blk_b, L), lambda i: (i, 0))],
        out_specs=pl.BlockSpec((blk_b, L), lambda i: (i, 0)),
    )(cnt0, cnt1)

    nrm2 = _sc_norm(dst2, rel2, nrow)

    w1r = weight1.reshape(R * N * 2, 16)
    a10, a11 = _sc_layer1(src2, rel2, dst2, nrm2, w1r, zeros2)

    w2cat = jnp.transpose(weight2, (1, 0, 2)).reshape(H, R * C)
    blk = 1000
    grid_e = N // blk
    z, xr = pl.pallas_call(
        _tc_dense_body,
        out_shape=[jax.ShapeDtypeStruct((N, R * C), _f32),
                   jax.ShapeDtypeStruct((N, C), _f32)],
        grid=(grid_e,),
        in_specs=[pl.BlockSpec((blk, L), lambda i: (i, 0)),
                  pl.BlockSpec((blk, L), lambda i: (i, 0)),
                  pl.BlockSpec((blk, H), lambda i: (i, 0)),
                  pl.BlockSpec((1, H), lambda i: (0, 0)),
                  pl.BlockSpec((H, R * C), lambda i: (0, 0)),
                  pl.BlockSpec((H, C), lambda i: (0, 0)),
                  pl.BlockSpec((1, C), lambda i: (0, 0))],
        out_specs=[pl.BlockSpec((blk, R * C), lambda i: (i, 0)),
                   pl.BlockSpec((blk, C), lambda i: (i, 0))],
    )(a10, a11, root1, bias1.reshape(1, H), w2cat, root2, bias2.reshape(1, C))
    zrows = z.reshape(N * R, C)

    p0, p1 = _sc_layer2(src2, rel2, dst2, nrm2, zrows, zeros2)

    blk_g = 1000
    grid_g = N // blk_g
    return pl.pallas_call(
        _tc_out_body,
        out_shape=jax.ShapeDtypeStruct((N, C), _f32),
        grid=(grid_g,),
        in_specs=[pl.BlockSpec((blk_g, C), lambda i: (i, 0)),
                  pl.BlockSpec((blk_g, C), lambda i: (i, 0)),
                  pl.BlockSpec((blk_g, C), lambda i: (i, 0))],
        out_specs=pl.BlockSpec((blk_g, C), lambda i: (i, 0)),
    )(p0, p1, xr)


# speedup vs baseline: 1.0752x; 1.0619x over previous
import jax
import jax.numpy as jnp
from jax import lax
from jax.experimental import pallas as pl
from jax.experimental.pallas import tpu as pltpu
from jax.experimental.pallas import tpu_sc as plsc

N = 100000
R = 8
H = 32
C = 16
E = 1600000

NC, NS, L = 2, 16, 16
NW = NC * NS
K = 512
KR = K // 128
E1 = 1638400
EPW = E1 // NW
EPT = E1 // NS
NP = 100096
NPS = NP // NS

_mesh = plsc.VectorSubcoreMesh(core_axis_name="c", subcore_axis_name="s")
_f32 = jnp.float32
_i32 = jnp.int32
_params = pltpu.CompilerParams(use_tc_tiling_on_sc=False,
                               needs_layout_passes=False)

_GDN = lax.GatherDimensionNumbers(offset_dims=(), collapsed_slice_dims=(0,),
                                  start_index_map=(0,))


def _bcast_lane(v, k):
    idx = jnp.full((L, 1), k, _i32)
    return lax.gather(v, idx, _GDN, (1,),
                      mode=lax.GatherScatterMode.PROMISE_IN_BOUNDS)


def _gather_fire(table_h, idxref, rows, bi, sem):
    for j in range(KR):
        pltpu.async_copy(table_h.at[idxref.at[j]],
                         rows.at[bi, pl.ds(j * 128, 128)], sem)


def _gather_drain(table_h, idxref, rows, bi, sem):
    for j in range(KR):
        pltpu.make_async_copy(table_h.at[idxref.at[j]],
                              rows.at[bi, pl.ds(j * 128, 128)], sem).wait()


def _scatter_fire(rows, bi, acc, dsti, r, sem):
    for j in range(KR):
        pltpu.async_copy(rows.at[bi, pl.ds(j * 128, 128)],
                         acc.at[dsti.at[r, j]], sem, add=True)


def _scatter_drain(rows, bi, acc, dsti, r, sem):
    for j in range(KR):
        pltpu.make_async_copy(rows.at[bi, pl.ds(j * 128, 128)],
                              acc.at[dsti.at[r, j]], sem).wait()


def _zero_and_barrier(zeros_h, acc, s):
    pltpu.sync_copy(zeros_h, acc.at[pl.ds(s * NPS, NPS)])
    plsc.subcore_barrier()


def _writeout(acc, c, s, out0_h, out1_h):
    plsc.subcore_barrier()

    @pl.when(c == 0)
    def _():
        pltpu.sync_copy(acc.at[pl.ds(s * NPS, NPS)],
                        out0_h.at[pl.ds(s * NPS, NPS)])

    @pl.when(c == 1)
    def _():
        pltpu.sync_copy(acc.at[pl.ds(s * NPS, NPS)],
                        out1_h.at[pl.ds(s * NPS, NPS)])


def _count_body(dst_h, rel_h, zeros_h, out0_h, out1_h, acc, dsti, reli, rows,
                isem, ssem):
    c = lax.axis_index("c")
    s = lax.axis_index("s")
    _zero_and_barrier(zeros_h, acc, s)
    wid = s * NC + c
    nch = EPW // K
    base_rb = wid * (EPW // 128)
    iot = lax.broadcasted_iota(_i32, (L,), 0)

    def idx_fire(i, r, sync=False):
        rb = base_rb + i * KR
        cp = pltpu.sync_copy if sync else (
            lambda a, b: pltpu.async_copy(a, b, isem))
        cp(dst_h.at[pl.ds(rb, KR)], dsti.at[r])
        cp(rel_h.at[pl.ds(rb, KR)], reli.at[r])

    def idx_drain(i, r):
        rb = base_rb + i * KR
        pltpu.make_async_copy(dst_h.at[pl.ds(rb, KR)], dsti.at[r], isem).wait()
        pltpu.make_async_copy(rel_h.at[pl.ds(rb, KR)], reli.at[r], isem).wait()

    def onehot(b, bi):
        def og(g, _):
            def ol(l, _):
                eb = g * 128 + l * 16
                relv = reli[b, g, pl.ds(l * 16, 16)]
                for k in range(16):
                    rows[bi, eb + k] = jnp.where(iot == relv[k], 1.0, 0.0)
                return 0
            return lax.fori_loop(0, 8, ol, 0)
        lax.fori_loop(0, KR, og, 0)

    idx_fire(0, 0, sync=True)
    idx_fire(1, 1)

    def outer(g, _):
        i0 = g * 4
        for b in range(4):
            i = i0 + b
            bi = b % 2

            @pl.when(i + 2 < nch)
            def _():
                idx_fire(i + 2, (b + 2) % 4)

            @pl.when(i >= 1)
            def _():
                idx_drain(i, b)
            onehot(b, bi)

            @pl.when(i >= 1)
            def _():
                _scatter_drain(rows, 1 - bi, acc, dsti, (b + 3) % 4, ssem)
            _scatter_fire(rows, bi, acc, dsti, b, ssem)
        return 0

    lax.fori_loop(0, nch // 4, outer, 0)
    _scatter_drain(rows, 1, acc, dsti, 3, ssem)
    _writeout(acc, c, s, out0_h, out1_h)


def _norm_body(dst_h, rel_h, nrow_h, out_h, dsti, reli, rows0,
               nout, gsem, isem, osem):
    c = lax.axis_index("c")
    s = lax.axis_index("s")
    wid = s * NC + c
    nch = EPW // K
    base_rb = wid * (EPW // 128)
    iot = lax.broadcasted_iota(_i32, (L,), 0)

    def idx_fire(i, r, sync=False):
        rb = base_rb + i * KR
        cp = pltpu.sync_copy if sync else (
            lambda a, b: pltpu.async_copy(a, b, isem))
        cp(dst_h.at[pl.ds(rb, KR)], dsti.at[r])
        cp(rel_h.at[pl.ds(rb, KR)], reli.at[r])

    def idx_drain(i, r):
        rb = base_rb + i * KR
        pltpu.make_async_copy(dst_h.at[pl.ds(rb, KR)], dsti.at[r], isem).wait()
        pltpu.make_async_copy(rel_h.at[pl.ds(rb, KR)], reli.at[r], isem).wait()

    def fire2(r, bi):
        _gather_fire(nrow_h, dsti.at[r], rows0, bi, gsem)

    def drain2(r, bi):
        _gather_drain(nrow_h, dsti.at[r], rows0, bi, gsem)

    def permute(b, bi):
        bvec = jnp.full((L,), bi, _i32)

        def pg(g, _):
            def plp(l, _):
                eb = g * 128 + l * 16
                relv = reli[b, g, pl.ds(l * 16, 16)]
                nv = plsc.load_gather(rows0, [bvec, eb + iot, relv])
                nout[bi, g, pl.ds(l * 16, 16)] = nv
                return 0
            return lax.fori_loop(0, 8, plp, 0)
        lax.fori_loop(0, KR, pg, 0)

    idx_fire(0, 0, sync=True)
    idx_fire(1, 1)
    fire2(0, 0)

    def outer(g, _):
        i0 = g * 4
        for b in range(4):
            i = i0 + b
            bi = b % 2

            @pl.when(i + 2 < nch)
            def _():
                idx_fire(i + 2, (b + 2) % 4)
            drain2(b, bi)

            @pl.when(i >= 1)
            def _():
                pltpu.make_async_copy(
                    nout.at[1 - bi],
                    out_h.at[pl.ds(base_rb + (i - 1) * KR, KR)], osem).wait()

            @pl.when(i + 1 < nch)
            def _():
                idx_drain(i + 1, (b + 1) % 4)
                fire2((b + 1) % 4, 1 - bi)
            permute(b, bi)
            pltpu.async_copy(nout.at[bi],
                             out_h.at[pl.ds(base_rb + i * KR, KR)], osem)
        return 0

    lax.fori_loop(0, nch // 4, outer, 0)
    pltpu.make_async_copy(nout.at[1],
                          out_h.at[pl.ds(base_rb + (nch - 1) * KR, KR)],
                          osem).wait()


def _edge_pass(src_h, rel_h, dst_h, nrm_h, tab_h, zeros_h, out0_h, out1_h,
               acc, srci, reli, dsti, nrmi, tidx, rows,
               gsem, isem, ssem, nch, base_rb, make_idx):
    c = lax.axis_index("c")
    s = lax.axis_index("s")
    _zero_and_barrier(zeros_h, acc, s)

    def idx_fire(i, r, sync=False):
        rb = base_rb + i * KR
        cp = pltpu.sync_copy if sync else (
            lambda a, b: pltpu.async_copy(a, b, isem))
        cp(src_h.at[pl.ds(rb, KR)], srci.at[r])
        cp(rel_h.at[pl.ds(rb, KR)], reli.at[r])
        cp(dst_h.at[pl.ds(rb, KR)], dsti.at[r])
        cp(nrm_h.at[pl.ds(rb, KR)], nrmi.at[r])

    def idx_drain(i, r):
        rb = base_rb + i * KR
        pltpu.make_async_copy(src_h.at[pl.ds(rb, KR)], srci.at[r], isem).wait()
        pltpu.make_async_copy(rel_h.at[pl.ds(rb, KR)], reli.at[r], isem).wait()
        pltpu.make_async_copy(dst_h.at[pl.ds(rb, KR)], dsti.at[r], isem).wait()
        pltpu.make_async_copy(nrm_h.at[pl.ds(rb, KR)], nrmi.at[r], isem).wait()

    def compute_tidx(r):
        def ig(g, _):
            def il(l, _):
                sv = srci[r, g, pl.ds(l * 16, 16)]
                rv = reli[r, g, pl.ds(l * 16, 16)]
                tidx[g, pl.ds(l * 16, 16)] = make_idx(sv, rv, c)
                return 0
            return lax.fori_loop(0, 8, il, 0)
        lax.fori_loop(0, KR, ig, 0)

    def scale(b, bi):
        def sg(g, _):
            def sl(l, _):
                eb = g * 128 + l * 16
                nv = nrmi[b, g, pl.ds(l * 16, 16)]
                for k in range(16):
                    rows[bi, eb + k] = rows[bi, eb + k] * _bcast_lane(nv, k)
                return 0
            return lax.fori_loop(0, 8, sl, 0)
        lax.fori_loop(0, KR, sg, 0)

    idx_fire(0, 0, sync=True)
    idx_fire(1, 1)
    compute_tidx(0)
    _gather_fire(tab_h, tidx, rows, 0, gsem)

    def outer(g, _):
        i0 = g * 4
        for b in range(4):
            i = i0 + b
            bi = b % 2

            @pl.when(i + 2 < nch)
            def _():
                idx_fire(i + 2, (b + 2) % 4)
            _gather_drain(tab_h, tidx, rows, bi, gsem)

            @pl.when(i >= 1)
            def _():
                _scatter_drain(rows, 1 - bi, acc, dsti, (b + 3) % 4, ssem)

            @pl.when(i + 1 < nch)
            def _():
                idx_drain(i + 1, (b + 1) % 4)
                compute_tidx((b + 1) % 4)
                _gather_fire(tab_h, tidx, rows, 1 - bi, gsem)
            scale(b, bi)
            _scatter_fire(rows, bi, acc, dsti, b, ssem)
        return 0

    lax.fori_loop(0, nch // 4, outer, 0)
    _scatter_drain(rows, 1, acc, dsti, 3, ssem)
    _writeout(acc, c, s, out0_h, out1_h)


def _layer1_body(src_h, rel_h, dst_h, nrm_h, w1_h, zeros_h, out0_h, out1_h,
                 acc, srci, reli, dsti, nrmi, tidx, rows, gsem, isem, ssem):
    s = lax.axis_index("s")
    _edge_pass(src_h, rel_h, dst_h, nrm_h, w1_h, zeros_h, out0_h, out1_h,
               acc, srci, reli, dsti, nrmi, tidx, rows, gsem, isem, ssem,
               nch=EPT // K, base_rb=s * (EPT // 128),
               make_idx=lambda sv, rv, c: (rv * N + sv) * 2 + c)


def _layer2_body(src_h, rel_h, dst_h, nrm_h, z_h, zeros_h, out0_h, out1_h,
                 acc, srci, reli, dsti, nrmi, tidx, rows, gsem, isem, ssem):
    c = lax.axis_index("c")
    s = lax.axis_index("s")
    wid = s * NC + c
    _edge_pass(src_h, rel_h, dst_h, nrm_h, z_h, zeros_h, out0_h, out1_h,
               acc, srci, reli, dsti, nrmi, tidx, rows, gsem, isem, ssem,
               nch=EPW // K, base_rb=wid * (EPW // 128),
               make_idx=lambda sv, rv, c: sv * R + rv)


_EDGE_SCRATCH = [
    pltpu.VMEM_SHARED((NP, L), _f32),
    pltpu.VMEM((4, KR, 128), _i32),
    pltpu.VMEM((4, KR, 128), _i32),
    pltpu.VMEM((4, KR, 128), _i32),
    pltpu.VMEM((4, KR, 128), _f32),
    pltpu.VMEM((KR, 128), _i32),
    pltpu.VMEM((2, K, L), _f32),
    pltpu.SemaphoreType.DMA,
    pltpu.SemaphoreType.DMA,
    pltpu.SemaphoreType.DMA,
]

_PAIR_OUT = [jax.ShapeDtypeStruct((NP, L), _f32),
             jax.ShapeDtypeStruct((NP, L), _f32)]


def _sc_count(dst2, rel2, zeros2):
    return pl.kernel(
        _count_body, out_type=_PAIR_OUT, mesh=_mesh, compiler_params=_params,
        scratch_types=[
            pltpu.VMEM_SHARED((NP, L), _f32),
            pltpu.VMEM((4, KR, 128), _i32),
            pltpu.VMEM((4, KR, 128), _i32),
            pltpu.VMEM((2, K, L), _f32),
            pltpu.SemaphoreType.DMA,
            pltpu.SemaphoreType.DMA,
        ],
    )(dst2, rel2, zeros2)


def _sc_norm(dst2, rel2, nrow):
    return pl.kernel(
        _norm_body, out_type=jax.ShapeDtypeStruct((E1 // 128, 128), _f32),
        mesh=_mesh, compiler_params=_params,
        scratch_types=[
            pltpu.VMEM((4, KR, 128), _i32),
            pltpu.VMEM((4, KR, 128), _i32),
            pltpu.VMEM((2, K, L), _f32),
            pltpu.VMEM((2, KR, 128), _f32),
            pltpu.SemaphoreType.DMA,
            pltpu.SemaphoreType.DMA,
            pltpu.SemaphoreType.DMA,
        ],
    )(dst2, rel2, nrow)


def _sc_layer1(src2, rel2, dst2, nrm2, w1r, zeros2):
    return pl.kernel(
        _layer1_body, out_type=_PAIR_OUT, mesh=_mesh, compiler_params=_params,
        scratch_types=_EDGE_SCRATCH,
    )(src2, rel2, dst2, nrm2, w1r, zeros2)


def _sc_layer2(src2, rel2, dst2, nrm2, zrows, zeros2):
    return pl.kernel(
        _layer2_body, out_type=_PAIR_OUT, mesh=_mesh, compiler_params=_params,
        scratch_types=_EDGE_SCRATCH,
    )(src2, rel2, dst2, nrm2, zrows, zeros2)


def _tc_norm_body(p0, p1, o):
    o[...] = 1.0 / jnp.maximum(p0[...] + p1[...], 1.0)


def _tc_dense_body(a0, a1, r1, b1, w2, r2, b2, z_o, xr_o):
    x = jnp.concatenate([a0[...], a1[...]], axis=1)
    x = jax.nn.relu(x + r1[...] + b1[...])
    z_o[...] = jnp.dot(x, w2[...], preferred_element_type=_f32)
    xr_o[...] = jnp.dot(x, r2[...], preferred_element_type=_f32) + b2[...]


def _tc_out_body(p0, p1, xr, o):
    o[...] = jax.nn.sigmoid(p0[...] + p1[...] + xr[...])


def kernel(edge_index, edge_type, weight1, root1, bias1, weight2, root2, bias2):
    src = edge_index[0].astype(_i32)
    dst = edge_index[1].astype(_i32)
    rel = edge_type.astype(_i32)
    pad = E1 - E
    src2 = jnp.concatenate([src, jnp.zeros((pad,), _i32)]).reshape(E1 // 128, 128)
    dst2 = jnp.concatenate([dst, jnp.full((pad,), N, _i32)]).reshape(E1 // 128, 128)
    rel2 = jnp.concatenate([rel, jnp.zeros((pad,), _i32)]).reshape(E1 // 128, 128)
    zeros2 = jnp.zeros((NPS, L), _f32)

    cnt0, cnt1 = _sc_count(dst2, rel2, zeros2)

    blk_b = 4352
    grid_b = NP // blk_b
    nrow = pl.pallas_call(
        _tc_norm_body,
        out_shape=jax.ShapeDtypeStruct((NP, L), _f32),
        grid=(grid_b,),
        in_specs=[pl.BlockSpec((blk_b, L), lambda i: (i, 0)),
                  pl.BlockSpec((blk_b, L), lambda i: (i, 0))],
        out_specs=pl.BlockSpec((blk_b, L), lambda i: (i, 0)),
    )(cnt0, cnt1)

    nrm2 = _sc_norm(dst2, rel2, nrow)

    w1r = weight1.reshape(R * N * 2, 16)
    a10, a11 = _sc_layer1(src2, rel2, dst2, nrm2, w1r, zeros2)

    w2cat = jnp.transpose(weight2, (1, 0, 2)).reshape(H, R * C)
    blk = 1000
    grid_e = N // blk
    z, xr = pl.pallas_call(
        _tc_dense_body,
        out_shape=[jax.ShapeDtypeStruct((N, R * C), _f32),
                   jax.ShapeDtypeStruct((N, C), _f32)],
        grid=(grid_e,),
        in_specs=[pl.BlockSpec((blk, L), lambda i: (i, 0)),
                  pl.BlockSpec((blk, L), lambda i: (i, 0)),
                  pl.BlockSpec((blk, H), lambda i: (i, 0)),
                  pl.BlockSpec((1, H), lambda i: (0, 0)),
                  pl.BlockSpec((H, R * C), lambda i: (0, 0)),
                  pl.BlockSpec((H, C), lambda i: (0, 0)),
                  pl.BlockSpec((1, C), lambda i: (0, 0))],
        out_specs=[pl.BlockSpec((blk, R * C), lambda i: (i, 0)),
                   pl.BlockSpec((blk, C), lambda i: (i, 0))],
    )(a10, a11, root1, bias1.reshape(1, H), w2cat, root2, bias2.reshape(1, C))
    zrows = z.reshape(N * R, C)

    p0, p1 = _sc_layer2(src2, rel2, dst2, nrm2, zrows, zeros2)

    blk_g = 1000
    grid_g = N // blk_g
    return pl.pallas_call(
        _tc_out_body,
        out_shape=jax.ShapeDtypeStruct((N, C), _f32),
        grid=(grid_g,),
        in_specs=[pl.BlockSpec((blk_g, C), lambda i: (i, 0)),
                  pl.BlockSpec((blk_g, C), lambda i: (i, 0)),
                  pl.BlockSpec((blk_g, C), lambda i: (i, 0))],
        out_specs=pl.BlockSpec((blk_g, C), lambda i: (i, 0)),
    )(p0, p1, xr)
